# baseline (device time: 97796 ns/iter reference)
import functools

import jax
import jax.numpy as jnp
from jax import lax
from jax.experimental import pallas as pl
from jax.experimental.pallas import tpu as pltpu

N_DEV = 4


def kernel(x, w_mat, scale_x, scale_w):
    m_per, k = x.shape
    k2, n_per = w_mat.shape
    assert k == k2
    half = m_per // 2

    x8 = x.astype(jnp.float8_e5m2)

    def body(x_ref, w_ref, sx_ref, sw_ref, out_ref,
             gath, wf32, w8, send_sems, recv_sems, w_sem):
        me = lax.axis_index("i")
        left = (me + N_DEV - 1) % N_DEV
        right = (me + 1) % N_DEV
        opp = (me + 2) % N_DEV

        barrier_sem = pltpu.get_barrier_semaphore()
        for nbr in (left, right):
            pl.semaphore_signal(barrier_sem, inc=1, device_id=(nbr,),
                                device_id_type=pl.DeviceIdType.MESH)
        pl.semaphore_wait(barrier_sem, 2)

        def copy(src, dst_start, rows, sidx, ridx, dev):
            return pltpu.make_async_remote_copy(
                src_ref=src,
                dst_ref=gath.at[pl.ds(dst_start, rows)],
                send_sem=send_sems.at[sidx],
                recv_sem=recv_sems.at[ridx],
                device_id=(dev,),
                device_id_type=pl.DeviceIdType.MESH,
            )

        own_to_r = copy(x_ref, me * m_per, m_per, 0, 0, right)
        own_to_l = copy(x_ref, me * m_per, m_per, 1, 1, left)
        own_to_r.start()
        own_to_l.start()

        w_dma = pltpu.make_async_copy(w_ref, wf32, w_sem)
        w_dma.start()
        w_dma.wait()
        w8[...] = wf32[...].astype(jnp.float8_e5m2)
        scale = sx_ref[0] * sw_ref[0]

        def block(origin, src):
            acc = jnp.dot(src, w8[...], preferred_element_type=jnp.float32)
            out_ref[pl.ds(origin * m_per, m_per), :] = acc * scale

        block(me, x_ref[...])

        own_from_l = copy(x_ref, left * m_per, m_per, 0, 0, left)
        own_from_r = copy(x_ref, right * m_per, m_per, 1, 1, right)

        own_from_l.wait_recv()
        fwd_to_r = copy(gath.at[pl.ds(left * m_per, half)],
                        left * m_per, half, 2, 2, right)
        fwd_to_r.start()

        own_from_r.wait_recv()
        fwd_to_l = copy(gath.at[pl.ds(right * m_per + half, half)],
                        right * m_per + half, half, 3, 3, left)
        fwd_to_l.start()

        block(left, gath[pl.ds(left * m_per, m_per), :])
        block(right, gath[pl.ds(right * m_per, m_per), :])

        fwd_from_l = copy(x_ref.at[pl.ds(0, half)], opp * m_per, half,
                          2, 2, left)
        fwd_from_r = copy(x_ref.at[pl.ds(0, half)], opp * m_per + half, half,
                          3, 3, right)
        fwd_from_l.wait_recv()
        fwd_from_r.wait_recv()

        block(opp, gath[pl.ds(opp * m_per, m_per), :])

        own_to_r.wait_send()
        own_to_l.wait_send()
        fwd_to_r.wait_send()
        fwd_to_l.wait_send()

        @functools.partial(pl.run_scoped,
                           second_barrier=pltpu.SemaphoreType.REGULAR)
        def _(second_barrier):
            for nbr in (left, right):
                pl.semaphore_signal(second_barrier, inc=1, device_id=(nbr,),
                                    device_id_type=pl.DeviceIdType.MESH)
            pl.semaphore_wait(second_barrier, 2)

    return pl.pallas_call(
        body,
        out_shape=jax.ShapeDtypeStruct((N_DEV * m_per, n_per), jnp.float32),
        in_specs=[
            pl.BlockSpec(memory_space=pltpu.VMEM),
            pl.BlockSpec(memory_space=pltpu.MemorySpace.HBM),
            pl.BlockSpec(memory_space=pltpu.SMEM),
            pl.BlockSpec(memory_space=pltpu.SMEM),
        ],
        out_specs=pl.BlockSpec(memory_space=pltpu.VMEM),
        scratch_shapes=[
            pltpu.VMEM((N_DEV * m_per, k), jnp.float8_e5m2),
            pltpu.VMEM((k, n_per), jnp.float32),
            pltpu.VMEM((k, n_per), jnp.float8_e5m2),
            pltpu.SemaphoreType.DMA((4,)),
            pltpu.SemaphoreType.DMA((4,)),
            pltpu.SemaphoreType.DMA,
        ],
        compiler_params=pltpu.CompilerParams(
            collective_id=0,
            vmem_limit_bytes=100 * 1024 * 1024,
        ),
    )(x8, w_mat, scale_x, scale_w)


# device time: 87895 ns/iter; 1.1126x vs baseline; 1.1126x over previous
import functools

import jax
import jax.numpy as jnp
from jax import lax
from jax.experimental import pallas as pl
from jax.experimental.pallas import tpu as pltpu

N_DEV = 4
N_SEG = 4


def kernel(x, w_mat, scale_x, scale_w):
    m_per, k = x.shape
    k2, n_per = w_mat.shape
    assert k == k2
    half = m_per // 2
    seg = m_per // N_SEG

    def body(x_ref, w_ref, sx_ref, sw_ref, out_ref,
             gath, xf32, wf32, w8, outv,
             send_sems, recv_sems, x_sems, w_sem, out_sems):
        me = lax.axis_index("i")
        left = (me + N_DEV - 1) % N_DEV
        right = (me + 1) % N_DEV
        opp = (me + 2) % N_DEV

        barrier_sem = pltpu.get_barrier_semaphore()
        for nbr in (left, right):
            pl.semaphore_signal(barrier_sem, inc=1, device_id=(nbr,),
                                device_id_type=pl.DeviceIdType.MESH)
        pl.semaphore_wait(barrier_sem, 2)

        x_dmas = [
            pltpu.make_async_copy(x_ref.at[pl.ds(i * seg, seg)],
                                  xf32.at[pl.ds(i * seg, seg)],
                                  x_sems.at[i])
            for i in range(N_SEG)
        ]
        for d in x_dmas:
            d.start()
        w_dma = pltpu.make_async_copy(w_ref, wf32, w_sem)
        w_dma.start()

        def copy(src, dst_start, rows, sidx, ridx, dev):
            return pltpu.make_async_remote_copy(
                src_ref=src,
                dst_ref=gath.at[pl.ds(dst_start, rows)],
                send_sem=send_sems.at[sidx],
                recv_sem=recv_sems.at[ridx],
                device_id=(dev,),
                device_id_type=pl.DeviceIdType.MESH,
            )

        sends = []
        for i in range(N_SEG):
            x_dmas[i].wait()
            gath[pl.ds(me * m_per + i * seg, seg), :] = (
                xf32[pl.ds(i * seg, seg), :].astype(jnp.float8_e5m2))
            src = gath.at[pl.ds(me * m_per + i * seg, seg)]
            to_r = copy(src, me * m_per + i * seg, seg, i, i, right)
            to_l = copy(src, me * m_per + i * seg, seg, N_SEG + i,
                        N_SEG + i, left)
            to_r.start()
            to_l.start()
            sends += [to_r, to_l]

        w_dma.wait()
        w8[...] = wf32[...].astype(jnp.float8_e5m2)
        scale = sx_ref[0] * sw_ref[0]

        out_dmas = []

        def block(origin, src):
            acc = jnp.dot(src, w8[...], preferred_element_type=jnp.float32)
            outv[pl.ds(origin * m_per, m_per), :] = acc * scale
            od = pltpu.make_async_copy(
                outv.at[pl.ds(origin * m_per, m_per)],
                out_ref.at[pl.ds(origin * m_per, m_per)],
                out_sems.at[len(out_dmas)])
            od.start()
            out_dmas.append(od)

        block(me, gath[pl.ds(me * m_per, m_per), :])

        for i in range(N_SEG):
            copy(gath.at[pl.ds(left * m_per + i * seg, seg)],
                 left * m_per + i * seg, seg, i, i, left).wait_recv()
        fwd_to_r = copy(gath.at[pl.ds(left * m_per, half)],
                        left * m_per, half, 2 * N_SEG, 2 * N_SEG, right)
        fwd_to_r.start()

        for i in range(N_SEG):
            copy(gath.at[pl.ds(right * m_per + i * seg, seg)],
                 right * m_per + i * seg, seg, N_SEG + i, N_SEG + i,
                 right).wait_recv()
        fwd_to_l = copy(gath.at[pl.ds(right * m_per + half, half)],
                        right * m_per + half, half, 2 * N_SEG + 1,
                        2 * N_SEG + 1, left)
        fwd_to_l.start()

        block(left, gath[pl.ds(left * m_per, m_per), :])
        block(right, gath[pl.ds(right * m_per, m_per), :])

        copy(gath.at[pl.ds(opp * m_per, half)], opp * m_per, half,
             2 * N_SEG, 2 * N_SEG, left).wait_recv()
        copy(gath.at[pl.ds(opp * m_per + half, half)], opp * m_per + half,
             half, 2 * N_SEG + 1, 2 * N_SEG + 1, right).wait_recv()

        block(opp, gath[pl.ds(opp * m_per, m_per), :])

        for s in sends:
            s.wait_send()
        fwd_to_r.wait_send()
        fwd_to_l.wait_send()
        for od in out_dmas:
            od.wait()

        @functools.partial(pl.run_scoped,
                           second_barrier=pltpu.SemaphoreType.REGULAR)
        def _(second_barrier):
            for nbr in (left, right):
                pl.semaphore_signal(second_barrier, inc=1, device_id=(nbr,),
                                    device_id_type=pl.DeviceIdType.MESH)
            pl.semaphore_wait(second_barrier, 2)

    n_sems = 2 * N_SEG + 2
    return pl.pallas_call(
        body,
        out_shape=jax.ShapeDtypeStruct((N_DEV * m_per, n_per), jnp.float32),
        in_specs=[
            pl.BlockSpec(memory_space=pltpu.MemorySpace.HBM),
            pl.BlockSpec(memory_space=pltpu.MemorySpace.HBM),
            pl.BlockSpec(memory_space=pltpu.SMEM),
            pl.BlockSpec(memory_space=pltpu.SMEM),
        ],
        out_specs=pl.BlockSpec(memory_space=pltpu.MemorySpace.HBM),
        scratch_shapes=[
            pltpu.VMEM((N_DEV * m_per, k), jnp.float8_e5m2),
            pltpu.VMEM((m_per, k), jnp.float32),
            pltpu.VMEM((k, n_per), jnp.float32),
            pltpu.VMEM((k, n_per), jnp.float8_e5m2),
            pltpu.VMEM((N_DEV * m_per, n_per), jnp.float32),
            pltpu.SemaphoreType.DMA((n_sems,)),
            pltpu.SemaphoreType.DMA((n_sems,)),
            pltpu.SemaphoreType.DMA((N_SEG,)),
            pltpu.SemaphoreType.DMA,
            pltpu.SemaphoreType.DMA((N_DEV,)),
        ],
        compiler_params=pltpu.CompilerParams(
            collective_id=0,
            vmem_limit_bytes=100 * 1024 * 1024,
        ),
    )(x, w_mat, scale_x, scale_w)


# device time: 85852 ns/iter; 1.1391x vs baseline; 1.0238x over previous
import functools

import jax
import jax.numpy as jnp
from jax import lax
from jax.experimental import pallas as pl
from jax.experimental.pallas import tpu as pltpu

N_DEV = 4
N_SEG = 8


def kernel(x, w_mat, scale_x, scale_w):
    m_per, k = x.shape
    k2, n_per = w_mat.shape
    assert k == k2
    half = m_per // 2
    seg = m_per // N_SEG
    top = list(range(N_SEG // 2))
    bot = list(range(N_SEG - 1, N_SEG // 2 - 1, -1))

    def body(x_ref, w_ref, sx_ref, sw_ref, out_ref,
             gath, xf32, wf32, w8, outv,
             send_sems, recv_sems, x_sems, w_sem, out_sems):
        me = lax.axis_index("i")
        left = (me + N_DEV - 1) % N_DEV
        right = (me + 1) % N_DEV
        opp = (me + 2) % N_DEV

        barrier_sem = pltpu.get_barrier_semaphore()
        for nbr in (left, right):
            pl.semaphore_signal(barrier_sem, inc=1, device_id=(nbr,),
                                device_id_type=pl.DeviceIdType.MESH)
        pl.semaphore_wait(barrier_sem, 2)

        x_dmas = {}
        for i in range(N_SEG):
            x_dmas[i] = pltpu.make_async_copy(
                x_ref.at[pl.ds(i * seg, seg)],
                xf32.at[pl.ds(i * seg, seg)],
                x_sems.at[i])
        dma_order = [s for pair in zip(top, bot) for s in pair]
        for i in dma_order:
            x_dmas[i].start()
        w_dma = pltpu.make_async_copy(w_ref, wf32, w_sem)
        w_dma.start()

        def copy(src, dst_start, rows, sidx, ridx, dev):
            return pltpu.make_async_remote_copy(
                src_ref=src,
                dst_ref=gath.at[pl.ds(dst_start, rows)],
                send_sem=send_sems.at[sidx],
                recv_sem=recv_sems.at[ridx],
                device_id=(dev,),
                device_id_type=pl.DeviceIdType.MESH,
            )

        sends = []
        cast_done = set()

        def send_seg(i, sidx, dev):
            if i not in cast_done:
                x_dmas[i].wait()
                gath[pl.ds(me * m_per + i * seg, seg), :] = (
                    xf32[pl.ds(i * seg, seg), :].astype(jnp.float8_e5m2))
                cast_done.add(i)
            s = copy(gath.at[pl.ds(me * m_per + i * seg, seg)],
                     me * m_per + i * seg, seg, sidx, sidx, dev)
            s.start()
            sends.append(s)

        for jr, jl in zip(top + bot[::-1], bot + top[::-1]):
            send_seg(jr, jr, right)
            send_seg(jl, N_SEG + jl, left)

        w_dma.wait()
        w8[...] = wf32[...].astype(jnp.float8_e5m2)
        scale = sx_ref[0] * sw_ref[0]

        out_dmas = []

        def block(origin, src):
            acc = jnp.dot(src, w8[...], preferred_element_type=jnp.float32)
            outv[pl.ds(origin * m_per, m_per), :] = acc * scale
            od = pltpu.make_async_copy(
                outv.at[pl.ds(origin * m_per, m_per)],
                out_ref.at[pl.ds(origin * m_per, m_per)],
                out_sems.at[len(out_dmas)])
            od.start()
            out_dmas.append(od)

        block(me, gath[pl.ds(me * m_per, m_per), :])

        def recv_seg(origin, i, ridx, dev):
            copy(gath.at[pl.ds(origin * m_per + i * seg, seg)],
                 origin * m_per + i * seg, seg, ridx, ridx, dev).wait_recv()

        for i in top:
            recv_seg(left, i, i, left)
        fwd_to_r = copy(gath.at[pl.ds(left * m_per, half)],
                        left * m_per, half, 2 * N_SEG, 2 * N_SEG, right)
        fwd_to_r.start()

        for i in bot:
            recv_seg(right, i, N_SEG + i, right)
        fwd_to_l = copy(gath.at[pl.ds(right * m_per + half, half)],
                        right * m_per + half, half, 2 * N_SEG + 1,
                        2 * N_SEG + 1, left)
        fwd_to_l.start()

        for i in bot:
            recv_seg(left, i, i, left)
        block(left, gath[pl.ds(left * m_per, m_per), :])
        for i in top:
            recv_seg(right, i, N_SEG + i, right)
        block(right, gath[pl.ds(right * m_per, m_per), :])

        copy(gath.at[pl.ds(opp * m_per, half)], opp * m_per, half,
             2 * N_SEG, 2 * N_SEG, left).wait_recv()
        copy(gath.at[pl.ds(opp * m_per + half, half)], opp * m_per + half,
             half, 2 * N_SEG + 1, 2 * N_SEG + 1, right).wait_recv()

        block(opp, gath[pl.ds(opp * m_per, m_per), :])

        for s in sends:
            s.wait_send()
        fwd_to_r.wait_send()
        fwd_to_l.wait_send()
        for od in out_dmas:
            od.wait()

        @functools.partial(pl.run_scoped,
                           second_barrier=pltpu.SemaphoreType.REGULAR)
        def _(second_barrier):
            for nbr in (left, right):
                pl.semaphore_signal(second_barrier, inc=1, device_id=(nbr,),
                                    device_id_type=pl.DeviceIdType.MESH)
            pl.semaphore_wait(second_barrier, 2)

    n_sems = 2 * N_SEG + 2
    return pl.pallas_call(
        body,
        out_shape=jax.ShapeDtypeStruct((N_DEV * m_per, n_per), jnp.float32),
        in_specs=[
            pl.BlockSpec(memory_space=pltpu.MemorySpace.HBM),
            pl.BlockSpec(memory_space=pltpu.MemorySpace.HBM),
            pl.BlockSpec(memory_space=pltpu.SMEM),
            pl.BlockSpec(memory_space=pltpu.SMEM),
        ],
        out_specs=pl.BlockSpec(memory_space=pltpu.MemorySpace.HBM),
        scratch_shapes=[
            pltpu.VMEM((N_DEV * m_per, k), jnp.float8_e5m2),
            pltpu.VMEM((m_per, k), jnp.float32),
            pltpu.VMEM((k, n_per), jnp.float32),
            pltpu.VMEM((k, n_per), jnp.float8_e5m2),
            pltpu.VMEM((N_DEV * m_per, n_per), jnp.float32),
            pltpu.SemaphoreType.DMA((n_sems,)),
            pltpu.SemaphoreType.DMA((n_sems,)),
            pltpu.SemaphoreType.DMA((N_SEG,)),
            pltpu.SemaphoreType.DMA,
            pltpu.SemaphoreType.DMA((N_DEV,)),
        ],
        compiler_params=pltpu.CompilerParams(
            collective_id=0,
            vmem_limit_bytes=100 * 1024 * 1024,
        ),
    )(x, w_mat, scale_x, scale_w)


# device time: 85211 ns/iter; 1.1477x vs baseline; 1.0075x over previous
import functools

import jax
import jax.numpy as jnp
from jax import lax
from jax.experimental import pallas as pl
from jax.experimental.pallas import tpu as pltpu

N_DEV = 4
N_SEG = 16


def kernel(x, w_mat, scale_x, scale_w):
    m_per, k = x.shape
    k2, n_per = w_mat.shape
    assert k == k2
    half = m_per // 2
    seg = m_per // N_SEG
    top = list(range(N_SEG // 2))
    bot = list(range(N_SEG - 1, N_SEG // 2 - 1, -1))

    def body(x_ref, w_ref, sx_ref, sw_ref, out_ref,
             gath, xf32, wf32, w8, outv,
             send_sems, recv_sems, x_sems, w_sem, out_sems):
        me = lax.axis_index("i")
        left = (me + N_DEV - 1) % N_DEV
        right = (me + 1) % N_DEV
        opp = (me + 2) % N_DEV

        barrier_sem = pltpu.get_barrier_semaphore()
        for nbr in (left, right):
            pl.semaphore_signal(barrier_sem, inc=1, device_id=(nbr,),
                                device_id_type=pl.DeviceIdType.MESH)
        pl.semaphore_wait(barrier_sem, 2)

        x_dmas = {}
        for i in range(N_SEG):
            x_dmas[i] = pltpu.make_async_copy(
                x_ref.at[pl.ds(i * seg, seg)],
                xf32.at[pl.ds(i * seg, seg)],
                x_sems.at[i])
        dma_order = [s for pair in zip(top, bot) for s in pair]
        for i in dma_order:
            x_dmas[i].start()
        w_dma = pltpu.make_async_copy(w_ref, wf32, w_sem)
        w_dma.start()

        def copy(src, dst_start, rows, sidx, ridx, dev):
            return pltpu.make_async_remote_copy(
                src_ref=src,
                dst_ref=gath.at[pl.ds(dst_start, rows)],
                send_sem=send_sems.at[sidx],
                recv_sem=recv_sems.at[ridx],
                device_id=(dev,),
                device_id_type=pl.DeviceIdType.MESH,
            )

        sends = []
        cast_done = set()

        def send_seg(i, sidx, dev):
            if i not in cast_done:
                x_dmas[i].wait()
                gath[pl.ds(me * m_per + i * seg, seg), :] = (
                    xf32[pl.ds(i * seg, seg), :].astype(jnp.float8_e5m2))
                cast_done.add(i)
            s = copy(gath.at[pl.ds(me * m_per + i * seg, seg)],
                     me * m_per + i * seg, seg, sidx, sidx, dev)
            s.start()
            sends.append(s)

        for jr, jl in zip(top + bot[::-1], bot + top[::-1]):
            send_seg(jr, jr, right)
            send_seg(jl, N_SEG + jl, left)

        w_dma.wait()
        w8[...] = wf32[...].astype(jnp.float8_e5m2)
        scale = sx_ref[0] * sw_ref[0]

        out_dmas = []

        def block(origin, src):
            acc = jnp.dot(src, w8[...], preferred_element_type=jnp.float32)
            outv[pl.ds(origin * m_per, m_per), :] = acc * scale
            od = pltpu.make_async_copy(
                outv.at[pl.ds(origin * m_per, m_per)],
                out_ref.at[pl.ds(origin * m_per, m_per)],
                out_sems.at[len(out_dmas)])
            od.start()
            out_dmas.append(od)

        block(me, gath[pl.ds(me * m_per, m_per), :])

        def recv_seg(origin, i, ridx, dev):
            copy(gath.at[pl.ds(origin * m_per + i * seg, seg)],
                 origin * m_per + i * seg, seg, ridx, ridx, dev).wait_recv()

        for i in top:
            recv_seg(left, i, i, left)
        fwd_to_r = copy(gath.at[pl.ds(left * m_per, half)],
                        left * m_per, half, 2 * N_SEG, 2 * N_SEG, right)
        fwd_to_r.start()

        for i in bot:
            recv_seg(right, i, N_SEG + i, right)
        fwd_to_l = copy(gath.at[pl.ds(right * m_per + half, half)],
                        right * m_per + half, half, 2 * N_SEG + 1,
                        2 * N_SEG + 1, left)
        fwd_to_l.start()

        for i in bot:
            recv_seg(left, i, i, left)
        block(left, gath[pl.ds(left * m_per, m_per), :])
        for i in top:
            recv_seg(right, i, N_SEG + i, right)
        block(right, gath[pl.ds(right * m_per, m_per), :])

        copy(gath.at[pl.ds(opp * m_per, half)], opp * m_per, half,
             2 * N_SEG, 2 * N_SEG, left).wait_recv()
        copy(gath.at[pl.ds(opp * m_per + half, half)], opp * m_per + half,
             half, 2 * N_SEG + 1, 2 * N_SEG + 1, right).wait_recv()

        block(opp, gath[pl.ds(opp * m_per, m_per), :])

        for s in sends:
            s.wait_send()
        fwd_to_r.wait_send()
        fwd_to_l.wait_send()
        for od in out_dmas:
            od.wait()

        @functools.partial(pl.run_scoped,
                           second_barrier=pltpu.SemaphoreType.REGULAR)
        def _(second_barrier):
            for nbr in (left, right):
                pl.semaphore_signal(second_barrier, inc=1, device_id=(nbr,),
                                    device_id_type=pl.DeviceIdType.MESH)
            pl.semaphore_wait(second_barrier, 2)

    n_sems = 2 * N_SEG + 2
    return pl.pallas_call(
        body,
        out_shape=jax.ShapeDtypeStruct((N_DEV * m_per, n_per), jnp.float32),
        in_specs=[
            pl.BlockSpec(memory_space=pltpu.MemorySpace.HBM),
            pl.BlockSpec(memory_space=pltpu.MemorySpace.HBM),
            pl.BlockSpec(memory_space=pltpu.SMEM),
            pl.BlockSpec(memory_space=pltpu.SMEM),
        ],
        out_specs=pl.BlockSpec(memory_space=pltpu.MemorySpace.HBM),
        scratch_shapes=[
            pltpu.VMEM((N_DEV * m_per, k), jnp.float8_e5m2),
            pltpu.VMEM((m_per, k), jnp.float32),
            pltpu.VMEM((k, n_per), jnp.float32),
            pltpu.VMEM((k, n_per), jnp.float8_e5m2),
            pltpu.VMEM((N_DEV * m_per, n_per), jnp.float32),
            pltpu.SemaphoreType.DMA((n_sems,)),
            pltpu.SemaphoreType.DMA((n_sems,)),
            pltpu.SemaphoreType.DMA((N_SEG,)),
            pltpu.SemaphoreType.DMA,
            pltpu.SemaphoreType.DMA((N_DEV,)),
        ],
        compiler_params=pltpu.CompilerParams(
            collective_id=0,
            vmem_limit_bytes=100 * 1024 * 1024,
        ),
    )(x, w_mat, scale_x, scale_w)
